# baseline (speedup 1.0000x reference)
import functools
import math

import jax
import jax.numpy as jnp
from jax import lax
from jax.experimental import pallas as pl
from jax.experimental.pallas import tpu as pltpu
from jax.experimental.pallas import tpu_sc as plsc

VOCAB = 1000000
EMB = 64
SCALE = math.sqrt(EMB)

NUM_WORKERS = 32
B_TOTAL = 4096 * 200
PER_W = B_TOTAL // NUM_WORKERS
CHUNK = 128
NCHUNK = PER_W // CHUNK
LANES = 16


def _make_kernel():
  mesh = plsc.VectorSubcoreMesh(core_axis_name="c", subcore_axis_name="s")

  @functools.partial(
      pl.kernel,
      mesh=mesh,
      out_type=jax.ShapeDtypeStruct((B_TOTAL, EMB), jnp.float32),
      compiler_params=pltpu.CompilerParams(use_tc_tiling_on_sc=False),
      scratch_types=[
          pltpu.VMEM((PER_W,), jnp.int32),
          pltpu.VMEM((CHUNK, EMB), jnp.float32),
          pltpu.SemaphoreType.DMA,
      ],
  )
  def emb_kernel(tokens_hbm, table_hbm, out_hbm, idx_v, rows_v, sem):
    wid = lax.axis_index("s") * 2 + lax.axis_index("c")
    base = wid * PER_W
    pltpu.sync_copy(tokens_hbm.at[pl.ds(base, PER_W)], idx_v)

    def chunk_body(c, carry):
      off = c * CHUNK
      pltpu.async_copy(
          table_hbm.at[idx_v.at[pl.ds(off, CHUNK)]], rows_v, sem).wait()

      def scale_body(j, carry2):
        for i in range(EMB // LANES):
          sl = pl.ds(i * LANES, LANES)
          rows_v[j, sl] = rows_v[j, sl] * SCALE
        return carry2

      lax.fori_loop(0, CHUNK, scale_body, 0, unroll=2)
      pltpu.sync_copy(rows_v, out_hbm.at[pl.ds(base + off, CHUNK)])
      return carry

    lax.fori_loop(0, NCHUNK, chunk_body, 0)

  return emb_kernel


_emb_kernel = _make_kernel()


def kernel(tokens, table):
  flat = tokens.reshape(-1).astype(jnp.int32)
  out = _emb_kernel(flat, table)
  return out.reshape(tokens.shape + (EMB,))


# speedup vs baseline: 1.1652x; 1.1652x over previous
import functools
import math

import jax
import jax.numpy as jnp
from jax import lax
from jax.experimental import pallas as pl
from jax.experimental.pallas import tpu as pltpu
from jax.experimental.pallas import tpu_sc as plsc

VOCAB = 1000000
EMB = 64
SCALE = math.sqrt(EMB)

NUM_WORKERS = 32
B_TOTAL = 4096 * 200
PER_W = B_TOTAL // NUM_WORKERS
CHUNK = 128
NCHUNK = PER_W // CHUNK
NBUF = 4
GROUP = 2 * NBUF
NBODY = NCHUNK // GROUP
LANES = 16


def _make_kernel():
  mesh = plsc.VectorSubcoreMesh(core_axis_name="c", subcore_axis_name="s")

  rows_scratch = [pltpu.VMEM((CHUNK, EMB), jnp.float32)
                  for _ in range(2 * NBUF)]
  gsem_scratch = [pltpu.SemaphoreType.DMA for _ in range(2 * NBUF)]

  @functools.partial(
      pl.kernel,
      mesh=mesh,
      out_type=jax.ShapeDtypeStruct((B_TOTAL, EMB), jnp.float32),
      compiler_params=pltpu.CompilerParams(use_tc_tiling_on_sc=False),
      scratch_types=[pltpu.VMEM((PER_W,), jnp.int32)]
      + rows_scratch
      + gsem_scratch
      + [pltpu.SemaphoreType.DMA, pltpu.SemaphoreType.DMA],
  )
  def emb_kernel(tokens_hbm, table_hbm, out_hbm, idx_v, *scratch):
    rows = scratch[:2 * NBUF]
    gsem = scratch[2 * NBUF:4 * NBUF]
    osem = scratch[4 * NBUF:]
    rows_ab = (rows[:NBUF], rows[NBUF:])
    gsem_ab = (gsem[:NBUF], gsem[NBUF:])

    wid = lax.axis_index("s") * 2 + lax.axis_index("c")
    base = wid * PER_W
    pltpu.sync_copy(tokens_hbm.at[pl.ds(base, PER_W)], idx_v)

    def scale_rows(buf):
      def scale_body(j, carry):
        for i in range(EMB // LANES):
          sl = pl.ds(i * LANES, LANES)
          buf[j, sl] = buf[j, sl] * SCALE
        return carry
      lax.fori_loop(0, CHUNK, scale_body, 0, unroll=2)

    def body(g, carry):
      goff = g * GROUP * CHUNK
      handles = [None] * 2
      for s in range(2):
        @pl.when(g > 0)
        def _(s=s):
          for b in range(NBUF):
            pltpu.make_async_copy(
                rows_ab[s][b], out_hbm.at[pl.ds(0, CHUNK)], osem[s]).wait()
        handles[s] = [
            pltpu.async_copy(
                table_hbm.at[idx_v.at[pl.ds(goff + (s * NBUF + b) * CHUNK,
                                            CHUNK)]],
                rows_ab[s][b], gsem_ab[s][b])
            for b in range(NBUF)
        ]
      for s in range(2):
        for b in range(NBUF):
          handles[s][b].wait()
          scale_rows(rows_ab[s][b])
          pltpu.async_copy(
              rows_ab[s][b],
              out_hbm.at[pl.ds(base + goff + (s * NBUF + b) * CHUNK, CHUNK)],
              osem[s])
      return carry

    lax.fori_loop(0, NBODY, body, 0)
    for s in range(2):
      for b in range(NBUF):
        pltpu.make_async_copy(
            rows_ab[s][b], out_hbm.at[pl.ds(0, CHUNK)], osem[s]).wait()

  return emb_kernel


_emb_kernel = _make_kernel()


def kernel(tokens, table):
  flat = tokens.reshape(-1).astype(jnp.int32)
  out = _emb_kernel(flat, table)
  return out.reshape(tokens.shape + (EMB,))
